# baseline (speedup 1.0000x reference)
import jax
import jax.numpy as jnp
import numpy as np
from jax.experimental import pallas as pl

_PARTS = [[0, 1, 2, 3, 20], [4, 5, 6, 7, 21, 22], [8, 9, 10, 11, 23, 24],
          [12, 13, 14, 15], [16, 17, 18, 19]]
_V_OUT = 25
_INV = [0] * _V_OUT
for _i, _p in enumerate(_PARTS):
    for _j in _p:
        _INV[_j] = _i


def _body(x_ref, o_ref):
    B, TV = x_ref.shape
    T = TV // 5
    W = T * _V_OUT
    NG = TV // T
    GW = W // NG
    lane = jax.lax.broadcasted_iota(jnp.int32, (1, GW), 1)
    groups = []
    for g in range(NG):
        t0 = (g * GW) // _V_OUT
        t1 = (g * GW + GW - 1) // _V_OUT
        c0, c1 = (5 * t0) // T, (5 * t1 + 4) // T
        gl = lane + g * GW
        t = gl // _V_OUT
        v = gl - t * _V_OUT
        inv_v = jnp.where(v < 20, v // 4, (v - 19) // 2)
        src = t * 5 + inv_v
        acc = None
        for c in range(c0, c1 + 1):
            lo = jnp.clip(src - c * T, 0, T - 1)
            idx = jnp.broadcast_to(lo, (B, GW))
            gat = jnp.take_along_axis(x_ref[:, c * T:(c + 1) * T], idx, axis=1)
            acc = gat if acc is None else jnp.where(src >= c * T, gat, acc)
        groups.append(acc)
    o_ref[...] = jnp.concatenate(groups, axis=1)


def kernel(x):
    N, C, T, V = x.shape
    R = N * C
    x2 = x.reshape(R, T * V)
    BR = 256
    while R % BR:
        BR //= 2
    out2 = pl.pallas_call(
        _body,
        grid=(R // BR,),
        in_specs=[pl.BlockSpec((BR, T * V), lambda i: (i, 0))],
        out_specs=pl.BlockSpec((BR, T * _V_OUT), lambda i: (i, 0)),
        out_shape=jax.ShapeDtypeStruct((R, T * _V_OUT), x.dtype),
    )(x2)
    return out2.reshape(N, C, T, _V_OUT)


# speedup vs baseline: 1.1183x; 1.1183x over previous
import jax
import jax.numpy as jnp
import numpy as np
from jax.experimental import pallas as pl

_PARTS = [[0, 1, 2, 3, 20], [4, 5, 6, 7, 21, 22], [8, 9, 10, 11, 23, 24],
          [12, 13, 14, 15], [16, 17, 18, 19]]
_V_OUT = 25
_INV = [0] * _V_OUT
for _i, _p in enumerate(_PARTS):
    for _j in _p:
        _INV[_j] = _i


def _onehot_g(T):
    W = T * _V_OUT
    g = np.zeros((T * 5, W), np.float32)
    for l in range(W):
        t, v = divmod(l, _V_OUT)
        g[5 * t + _INV[v], l] = 1.0
    return g


def _body(x_ref, g_ref, o_ref):
    xv = x_ref[...]
    hi = xv.astype(jnp.bfloat16)
    r1 = xv - hi.astype(jnp.float32)
    mid = r1.astype(jnp.bfloat16)
    lo = (r1 - mid.astype(jnp.float32)).astype(jnp.bfloat16)
    g = g_ref[...]
    acc = jnp.dot(hi, g, preferred_element_type=jnp.float32)
    acc = acc + jnp.dot(mid, g, preferred_element_type=jnp.float32)
    acc = acc + jnp.dot(lo, g, preferred_element_type=jnp.float32)
    o_ref[...] = acc


def kernel(x):
    N, C, T, V = x.shape
    R = N * C
    W = T * _V_OUT
    x2 = x.reshape(R, T * V)
    g = jnp.asarray(_onehot_g(T), dtype=jnp.bfloat16)
    BR = 512
    while R % BR:
        BR //= 2
    out2 = pl.pallas_call(
        _body,
        grid=(R // BR,),
        in_specs=[
            pl.BlockSpec((BR, T * V), lambda i: (i, 0)),
            pl.BlockSpec((T * V, W), lambda i: (0, 0)),
        ],
        out_specs=pl.BlockSpec((BR, W), lambda i: (i, 0)),
        out_shape=jax.ShapeDtypeStruct((R, W), x.dtype),
    )(x2, g)
    return out2.reshape(N, C, T, _V_OUT)


# speedup vs baseline: 12.7351x; 11.3877x over previous
import jax
import jax.numpy as jnp
from jax.experimental import pallas as pl

_PARTS = [[0, 1, 2, 3, 20], [4, 5, 6, 7, 21, 22], [8, 9, 10, 11, 23, 24],
          [12, 13, 14, 15], [16, 17, 18, 19]]
_V_OUT = 25
_PERM = [v for p in _PARTS for v in p]
_BOUNDS = []
_acc = 0
for _p in _PARTS:
    _acc += len(_p)
    _BOUNDS.append(_acc)


def _src_index(j):
    p = (j >= _BOUNDS[0]).astype(jnp.int32)
    for b in _BOUNDS[1:-1]:
        p = p + (j >= b).astype(jnp.int32)
    return p


def _dst_index(j):
    v = jnp.int32(0)
    for k, pv in enumerate(_PERM):
        v = jnp.where(j == k, jnp.int32(pv), v)
    return v


def _body(x_ref, o_ref):
    o_ref[...] = x_ref[...]


def kernel(x):
    N, C, T, V = x.shape
    xt = jnp.transpose(x, (0, 3, 1, 2))
    out_t = pl.pallas_call(
        _body,
        grid=(_V_OUT,),
        in_specs=[pl.BlockSpec((N, 1, C, T), lambda j: (0, _src_index(j), 0, 0))],
        out_specs=pl.BlockSpec((N, 1, C, T), lambda j: (0, _dst_index(j), 0, 0)),
        out_shape=jax.ShapeDtypeStruct((N, _V_OUT, C, T), x.dtype),
    )(xt)
    return jnp.transpose(out_t, (0, 2, 3, 1))


# speedup vs baseline: 14.3102x; 1.1237x over previous
import jax
import jax.numpy as jnp
from jax.experimental import pallas as pl
from jax.experimental.pallas import tpu as pltpu

_PARTS = [[0, 1, 2, 3, 20], [4, 5, 6, 7, 21, 22], [8, 9, 10, 11, 23, 24],
          [12, 13, 14, 15], [16, 17, 18, 19]]
_V_OUT = 25


def _body(x_hbm, o_hbm, vbuf, in_sems, out_sem):
    in_cps = []
    for i in range(5):
        cp = pltpu.make_async_copy(x_hbm.at[:, i], vbuf.at[i], in_sems.at[i])
        cp.start()
        in_cps.append(cp)
    out_cps = []
    for pi, part in enumerate(_PARTS):
        in_cps[pi].wait()
        for v in part:
            cp = pltpu.make_async_copy(vbuf.at[pi], o_hbm.at[:, v], out_sem)
            cp.start()
            out_cps.append(cp)
    for cp in out_cps:
        cp.wait()


def kernel(x):
    N, C, T, V = x.shape
    xt = jnp.transpose(x, (0, 3, 1, 2))
    out_t = pl.pallas_call(
        _body,
        in_specs=[pl.BlockSpec(memory_space=pl.ANY)],
        out_specs=pl.BlockSpec(memory_space=pl.ANY),
        out_shape=jax.ShapeDtypeStruct((N, _V_OUT, C, T), x.dtype),
        scratch_shapes=[
            pltpu.VMEM((V, N, C, T), jnp.float32),
            pltpu.SemaphoreType.DMA((V,)),
            pltpu.SemaphoreType.DMA,
        ],
    )(xt)
    return jnp.transpose(out_t, (0, 2, 3, 1))
